# baseline (speedup 1.0000x reference)
import functools

import jax
import jax.numpy as jnp
from jax import lax
from jax.experimental import pallas as pl
from jax.experimental.pallas import tpu as pltpu
from jax.experimental.pallas import tpu_sc as plsc

N = 10000
NP = 10240
TRASH = N
E = 320000
D = 128
NC = 2
NS = 16
NW = NC * NS
EPT = E // NW
CH = 64
SB = 16
NSB = 10
EPT_PAD = NSB * SB * CH
RPW = NP // NS
IRN = RPW // CH

_mesh = plsc.VectorSubcoreMesh(core_axis_name="c", subcore_axis_name="s")


@functools.partial(
    pl.kernel,
    out_type=jax.ShapeDtypeStruct((NC, NP, 16), jnp.float32),
    mesh=_mesh,
    scratch_types=[
        pltpu.VMEM_SHARED((NP, 16), jnp.float32),
        pltpu.VMEM((SB, CH), jnp.int32),
        pltpu.VMEM((CH, 16), jnp.float32),
        pltpu.VMEM((CH, 16), jnp.float32),
    ],
)
def _deg_kernel(col_hbm, ones_hbm, zeros_hbm, out_hbm, acc, coli, ones, zbuf):
    c = lax.axis_index("c")
    s = lax.axis_index("s")
    w = s * NC + c
    pltpu.sync_copy(ones_hbm, ones)
    pltpu.sync_copy(zeros_hbm, zbuf)
    base = s * RPW

    @pl.loop(0, IRN)
    def _(r):
        pltpu.sync_copy(zbuf, acc.at[pl.ds(base + r * CH, CH)])

    plsc.subcore_barrier()

    @pl.loop(0, NSB)
    def _(sb):
        pltpu.sync_copy(col_hbm.at[w, pl.ds(sb * SB, SB)], coli)

        @pl.loop(0, SB)
        def _(k):
            pltpu.sync_copy(ones, acc.at[coli.at[k]], add=True)

    plsc.subcore_barrier()

    @pl.loop(0, IRN)
    def _(r):
        pltpu.sync_copy(acc.at[pl.ds(base + r * CH, CH)], zbuf)
        pltpu.sync_copy(zbuf, out_hbm.at[c, pl.ds(base + r * CH, CH)])


NCHT = NSB * SB


@functools.partial(
    pl.kernel,
    out_type=jax.ShapeDtypeStruct((NC, NP, D), jnp.float32),
    mesh=_mesh,
    scratch_types=[
        pltpu.VMEM_SHARED((NP, D), jnp.float32),
        pltpu.VMEM((SB, CH), jnp.int32),
        pltpu.VMEM((SB, CH), jnp.int32),
        pltpu.VMEM((SB, CH), jnp.int32),
        pltpu.VMEM((SB, CH), jnp.int32),
        pltpu.VMEM((2, CH, D), jnp.float32),
        pltpu.SemaphoreType.DMA,
        pltpu.SemaphoreType.DMA,
        pltpu.SemaphoreType.DMA,
    ],
)
def _agg_kernel(g_hbm, row_hbm, col_hbm, out_hbm, acc, rowi0, rowi1,
                coli0, coli1, dbuf, gsa, gsb, ssem):
    c = lax.axis_index("c")
    s = lax.axis_index("s")
    w = s * NC + c
    base = s * RPW
    HC = CH // 2

    def issue_gather(ri, k, slot):
        pltpu.async_copy(g_hbm.at[ri.at[k, pl.ds(0, HC)]],
                         dbuf.at[slot, pl.ds(0, HC)], gsa)
        pltpu.async_copy(g_hbm.at[ri.at[k, pl.ds(HC, HC)]],
                         dbuf.at[slot, pl.ds(HC, HC)], gsb)

    def wait_gather(ri, k, slot):
        pltpu.make_async_copy(g_hbm.at[ri.at[k, pl.ds(0, HC)]],
                              dbuf.at[slot, pl.ds(0, HC)], gsa).wait()
        pltpu.make_async_copy(g_hbm.at[ri.at[k, pl.ds(HC, HC)]],
                              dbuf.at[slot, pl.ds(HC, HC)], gsb).wait()

    @pl.loop(0, IRN)
    def _(r):
        pltpu.sync_copy(g_hbm.at[pl.ds(base + r * CH, CH)], dbuf.at[0])
        pltpu.sync_copy(dbuf.at[0], acc.at[pl.ds(base + r * CH, CH)])

    plsc.subcore_barrier()

    pltpu.sync_copy(row_hbm.at[w, pl.ds(0, SB)], rowi0)
    pltpu.sync_copy(col_hbm.at[w, pl.ds(0, SB)], coli0)
    issue_gather(rowi0, 0, 0)
    pltpu.sync_copy(row_hbm.at[w, pl.ds(SB, SB)], rowi1)
    pltpu.sync_copy(col_hbm.at[w, pl.ds(SB, SB)], coli1)

    for sb in range(NSB):
        ri, ci = (rowi0, coli0) if sb % 2 == 0 else (rowi1, coli1)
        rn, cn = (rowi1, coli1) if sb % 2 == 0 else (rowi0, coli0)

        @pl.loop(0, SB - 1)
        def _(k):
            slot = lax.rem(k, 2)
            wait_gather(ri, k, slot)
            pltpu.async_copy(dbuf.at[slot], acc.at[ci.at[k]], ssem, add=True)

            @pl.when(k > 0)
            def _():
                pltpu.make_async_copy(
                    dbuf.at[1 - slot], acc.at[ci.at[k]], ssem).wait()

            issue_gather(ri, k + 1, 1 - slot)

        wait_gather(ri, SB - 1, 1)
        pltpu.async_copy(dbuf.at[1], acc.at[ci.at[SB - 1]], ssem, add=True)
        pltpu.make_async_copy(dbuf.at[0], acc.at[ci.at[0]], ssem).wait()
        if sb + 1 < NSB:
            issue_gather(rn, 0, 0)
        pltpu.make_async_copy(dbuf.at[1], acc.at[ci.at[0]], ssem).wait()
        if sb + 2 < NSB:
            pltpu.sync_copy(row_hbm.at[w, pl.ds((sb + 2) * SB, SB)], ri)
            pltpu.sync_copy(col_hbm.at[w, pl.ds((sb + 2) * SB, SB)], ci)

    plsc.subcore_barrier()

    @pl.loop(0, IRN)
    def _(r):
        pltpu.sync_copy(acc.at[pl.ds(base + r * CH, CH)], dbuf.at[0])
        pltpu.sync_copy(dbuf.at[0], out_hbm.at[c, pl.ds(base + r * CH, CH)])


_BLK = 1024
_GRID = NP // _BLK


def _dinv_block(pd_blk):
    deg = pd_blk[0, :, 0:1] + pd_blk[1, :, 0:1] + 1.0
    return lax.rsqrt(deg)


def _tc_pre_body(x_ref, w_ref, pd_ref, o_ref):
    h = lax.dot_general(x_ref[...], w_ref[...], (((1,), (1,)), ((), ())),
                        precision=lax.Precision.HIGHEST)
    o_ref[...] = h * _dinv_block(pd_ref[...])


def _tc_mid_body(p_ref, g_ref, b_ref, w_ref, pd_ref, o_ref):
    dinv = _dinv_block(pd_ref[...])
    sg = p_ref[0] + p_ref[1] - g_ref[...]
    o = jnp.maximum(sg * dinv + b_ref[...], 0.0)
    h = lax.dot_general(o, w_ref[...], (((1,), (1,)), ((), ())),
                        precision=lax.Precision.HIGHEST)
    o_ref[...] = h * dinv


def _tc_last_body(p_ref, g_ref, b_ref, pd_ref, o_ref):
    dinv = _dinv_block(pd_ref[...])
    sg = p_ref[0] + p_ref[1] - g_ref[...]
    o_ref[...] = sg * dinv + b_ref[...]


_pd_spec = pl.BlockSpec((2, _BLK, 16), lambda i: (0, i, 0))
_x_spec = pl.BlockSpec((_BLK, D), lambda i: (i, 0))
_w_spec = pl.BlockSpec((D, D), lambda i: (0, 0))
_b_spec = pl.BlockSpec((1, D), lambda i: (0, 0))
_p_spec = pl.BlockSpec((2, _BLK, D), lambda i: (0, i, 0))

_tc_pre = pl.pallas_call(
    _tc_pre_body,
    grid=(_GRID,),
    in_specs=[_x_spec, _w_spec, _pd_spec],
    out_specs=_x_spec,
    out_shape=jax.ShapeDtypeStruct((NP, D), jnp.float32),
)

_tc_mid = pl.pallas_call(
    _tc_mid_body,
    grid=(_GRID,),
    in_specs=[_p_spec, _x_spec, _b_spec, _w_spec, _pd_spec],
    out_specs=_x_spec,
    out_shape=jax.ShapeDtypeStruct((NP, D), jnp.float32),
)

_tc_last = pl.pallas_call(
    _tc_last_body,
    grid=(_GRID,),
    in_specs=[_p_spec, _x_spec, _b_spec, _pd_spec],
    out_specs=_x_spec,
    out_shape=jax.ShapeDtypeStruct((NP, D), jnp.float32),
)


def kernel(x, edge_index, W1, b1, W2, b2, W3, b3):
    row = edge_index[0].astype(jnp.int32)
    col = edge_index[1].astype(jnp.int32)
    pad = EPT_PAD - EPT
    row_t = jnp.pad(row.reshape(NW, EPT), ((0, 0), (0, pad)),
                    constant_values=0).reshape(NW, NSB * SB, CH)
    col_t = jnp.pad(col.reshape(NW, EPT), ((0, 0), (0, pad)),
                    constant_values=TRASH).reshape(NW, NSB * SB, CH)
    x_p = jnp.pad(x, ((0, NP - N), (0, 0)))
    ones16 = jnp.ones((CH, 16), jnp.float32)
    zeros16 = jnp.zeros((CH, 16), jnp.float32)
    b1r = b1.reshape(1, D)
    b2r = b2.reshape(1, D)
    b3r = b3.reshape(1, D)

    pd = _deg_kernel(col_t, ones16, zeros16)
    g1 = _tc_pre(x_p, W1, pd)
    p1 = _agg_kernel(g1, row_t, col_t)
    g2 = _tc_mid(p1, g1, b1r, W2, pd)
    p2 = _agg_kernel(g2, row_t, col_t)
    g3 = _tc_mid(p2, g2, b2r, W3, pd)
    p3 = _agg_kernel(g3, row_t, col_t)
    out = _tc_last(p3, g3, b3r, pd)
    return out[:N]


# speedup vs baseline: 1.0017x; 1.0017x over previous
import functools

import jax
import jax.numpy as jnp
from jax import lax
from jax.experimental import pallas as pl
from jax.experimental.pallas import tpu as pltpu
from jax.experimental.pallas import tpu_sc as plsc

N = 10000
NP = 10240
TRASH = N
E = 320000
D = 128
NC = 2
NS = 16
NW = NC * NS
EPT = E // NW
CH = 64
SB = 16
NSB = 10
EPT_PAD = NSB * SB * CH
RPW = NP // NS
IRN = RPW // CH

_mesh = plsc.VectorSubcoreMesh(core_axis_name="c", subcore_axis_name="s")


@functools.partial(
    pl.kernel,
    out_type=jax.ShapeDtypeStruct((NC, NP, 16), jnp.float32),
    mesh=_mesh,
    scratch_types=[
        pltpu.VMEM_SHARED((NP, 16), jnp.float32),
        pltpu.VMEM((SB, CH), jnp.int32),
        pltpu.VMEM((CH, 16), jnp.float32),
        pltpu.VMEM((CH, 16), jnp.float32),
    ],
)
def _deg_kernel(col_hbm, ones_hbm, zeros_hbm, out_hbm, acc, coli, ones, zbuf):
    c = lax.axis_index("c")
    s = lax.axis_index("s")
    w = s * NC + c
    pltpu.sync_copy(ones_hbm, ones)
    pltpu.sync_copy(zeros_hbm, zbuf)
    base = s * RPW

    @pl.loop(0, IRN)
    def _(r):
        pltpu.sync_copy(zbuf, acc.at[pl.ds(base + r * CH, CH)])

    plsc.subcore_barrier()

    @pl.loop(0, NSB)
    def _(sb):
        pltpu.sync_copy(col_hbm.at[w, pl.ds(sb * SB, SB)], coli)

        @pl.loop(0, SB)
        def _(k):
            pltpu.sync_copy(ones, acc.at[coli.at[k]], add=True)

    plsc.subcore_barrier()

    @pl.loop(0, IRN)
    def _(r):
        pltpu.sync_copy(acc.at[pl.ds(base + r * CH, CH)], zbuf)
        pltpu.sync_copy(zbuf, out_hbm.at[c, pl.ds(base + r * CH, CH)])


NCHT = NSB * SB


@functools.partial(
    pl.kernel,
    out_type=jax.ShapeDtypeStruct((NC, NP, D), jnp.float32),
    mesh=_mesh,
    scratch_types=[
        pltpu.VMEM_SHARED((NP, D), jnp.float32),
        pltpu.VMEM((SB, CH), jnp.int32),
        pltpu.VMEM((SB, CH), jnp.int32),
        pltpu.VMEM((SB, CH), jnp.int32),
        pltpu.VMEM((SB, CH), jnp.int32),
        pltpu.VMEM((2, CH, D), jnp.float32),
        pltpu.SemaphoreType.DMA,
        pltpu.SemaphoreType.DMA,
    ],
)
def _agg_kernel(g_hbm, row_hbm, col_hbm, out_hbm, acc, rowi0, rowi1,
                coli0, coli1, dbuf, gsa, gsb):
    c = lax.axis_index("c")
    s = lax.axis_index("s")
    w = s * NC + c
    base = s * RPW
    HC = CH // 2

    def issue_gather(ri, k, slot):
        pltpu.async_copy(g_hbm.at[ri.at[k, pl.ds(0, HC)]],
                         dbuf.at[slot, pl.ds(0, HC)], gsa)
        pltpu.async_copy(g_hbm.at[ri.at[k, pl.ds(HC, HC)]],
                         dbuf.at[slot, pl.ds(HC, HC)], gsb)

    def wait_gather(ri, k, slot):
        pltpu.make_async_copy(g_hbm.at[ri.at[k, pl.ds(0, HC)]],
                              dbuf.at[slot, pl.ds(0, HC)], gsa).wait()
        pltpu.make_async_copy(g_hbm.at[ri.at[k, pl.ds(HC, HC)]],
                              dbuf.at[slot, pl.ds(HC, HC)], gsb).wait()

    @pl.loop(0, IRN)
    def _(r):
        pltpu.sync_copy(g_hbm.at[pl.ds(base + r * CH, CH)], dbuf.at[0])
        pltpu.sync_copy(dbuf.at[0], acc.at[pl.ds(base + r * CH, CH)])

    plsc.subcore_barrier()

    pltpu.sync_copy(row_hbm.at[w, pl.ds(0, SB)], rowi0)
    pltpu.sync_copy(col_hbm.at[w, pl.ds(0, SB)], coli0)
    issue_gather(rowi0, 0, 0)
    pltpu.sync_copy(row_hbm.at[w, pl.ds(SB, SB)], rowi1)
    pltpu.sync_copy(col_hbm.at[w, pl.ds(SB, SB)], coli1)

    for sb in range(NSB):
        ri, ci = (rowi0, coli0) if sb % 2 == 0 else (rowi1, coli1)
        rn, cn = (rowi1, coli1) if sb % 2 == 0 else (rowi0, coli0)

        @pl.loop(0, SB - 1)
        def _(k):
            slot = lax.rem(k, 2)
            wait_gather(ri, k, slot)
            issue_gather(ri, k + 1, 1 - slot)
            pltpu.sync_copy(dbuf.at[slot], acc.at[ci.at[k]], add=True)

        wait_gather(ri, SB - 1, 1)
        if sb + 1 < NSB:
            issue_gather(rn, 0, 0)
        pltpu.sync_copy(dbuf.at[1], acc.at[ci.at[SB - 1]], add=True)
        if sb + 2 < NSB:
            pltpu.sync_copy(row_hbm.at[w, pl.ds((sb + 2) * SB, SB)], ri)
            pltpu.sync_copy(col_hbm.at[w, pl.ds((sb + 2) * SB, SB)], ci)

    plsc.subcore_barrier()

    @pl.loop(0, IRN)
    def _(r):
        pltpu.sync_copy(acc.at[pl.ds(base + r * CH, CH)], dbuf.at[0])
        pltpu.sync_copy(dbuf.at[0], out_hbm.at[c, pl.ds(base + r * CH, CH)])


_BLK = 1024
_GRID = NP // _BLK


def _dinv_block(pd_blk):
    deg = pd_blk[0, :, 0:1] + pd_blk[1, :, 0:1] + 1.0
    return lax.rsqrt(deg)


def _tc_pre_body(x_ref, w_ref, pd_ref, o_ref):
    h = lax.dot_general(x_ref[...], w_ref[...], (((1,), (1,)), ((), ())),
                        precision=lax.Precision.HIGHEST)
    o_ref[...] = h * _dinv_block(pd_ref[...])


def _tc_mid_body(p_ref, g_ref, b_ref, w_ref, pd_ref, o_ref):
    dinv = _dinv_block(pd_ref[...])
    sg = p_ref[0] + p_ref[1] - g_ref[...]
    o = jnp.maximum(sg * dinv + b_ref[...], 0.0)
    h = lax.dot_general(o, w_ref[...], (((1,), (1,)), ((), ())),
                        precision=lax.Precision.HIGHEST)
    o_ref[...] = h * dinv


def _tc_last_body(p_ref, g_ref, b_ref, pd_ref, o_ref):
    dinv = _dinv_block(pd_ref[...])
    sg = p_ref[0] + p_ref[1] - g_ref[...]
    o_ref[...] = sg * dinv + b_ref[...]


_pd_spec = pl.BlockSpec((2, _BLK, 16), lambda i: (0, i, 0))
_x_spec = pl.BlockSpec((_BLK, D), lambda i: (i, 0))
_w_spec = pl.BlockSpec((D, D), lambda i: (0, 0))
_b_spec = pl.BlockSpec((1, D), lambda i: (0, 0))
_p_spec = pl.BlockSpec((2, _BLK, D), lambda i: (0, i, 0))

_tc_pre = pl.pallas_call(
    _tc_pre_body,
    grid=(_GRID,),
    in_specs=[_x_spec, _w_spec, _pd_spec],
    out_specs=_x_spec,
    out_shape=jax.ShapeDtypeStruct((NP, D), jnp.float32),
)

_tc_mid = pl.pallas_call(
    _tc_mid_body,
    grid=(_GRID,),
    in_specs=[_p_spec, _x_spec, _b_spec, _w_spec, _pd_spec],
    out_specs=_x_spec,
    out_shape=jax.ShapeDtypeStruct((NP, D), jnp.float32),
)

_tc_last = pl.pallas_call(
    _tc_last_body,
    grid=(_GRID,),
    in_specs=[_p_spec, _x_spec, _b_spec, _pd_spec],
    out_specs=_x_spec,
    out_shape=jax.ShapeDtypeStruct((NP, D), jnp.float32),
)


def kernel(x, edge_index, W1, b1, W2, b2, W3, b3):
    row = edge_index[0].astype(jnp.int32)
    col = edge_index[1].astype(jnp.int32)
    pad = EPT_PAD - EPT
    row_t = jnp.pad(row.reshape(NW, EPT), ((0, 0), (0, pad)),
                    constant_values=0).reshape(NW, NSB * SB, CH)
    col_t = jnp.pad(col.reshape(NW, EPT), ((0, 0), (0, pad)),
                    constant_values=TRASH).reshape(NW, NSB * SB, CH)
    x_p = jnp.pad(x, ((0, NP - N), (0, 0)))
    ones16 = jnp.ones((CH, 16), jnp.float32)
    zeros16 = jnp.zeros((CH, 16), jnp.float32)
    b1r = b1.reshape(1, D)
    b2r = b2.reshape(1, D)
    b3r = b3.reshape(1, D)

    pd = _deg_kernel(col_t, ones16, zeros16)
    g1 = _tc_pre(x_p, W1, pd)
    p1 = _agg_kernel(g1, row_t, col_t)
    g2 = _tc_mid(p1, g1, b1r, W2, pd)
    p2 = _agg_kernel(g2, row_t, col_t)
    g3 = _tc_mid(p2, g2, b2r, W3, pd)
    p3 = _agg_kernel(g3, row_t, col_t)
    out = _tc_last(p3, g3, b3r, pd)
    return out[:N]
